# baseline (speedup 1.0000x reference)
import functools

import jax
import jax.numpy as jnp
from jax import lax
from jax.experimental import pallas as pl
from jax.experimental.pallas import tpu as pltpu
from jax.experimental.pallas import tpu_sc as plsc

_MEAN = 3.5
_CHUNK = 128


@functools.partial(jax.jit, static_argnames=("batch", "k"))
def _biased_mf_sc(user_ids, item_ids, bu, bv, U, V, *, batch, k):
    info = plsc.get_sparse_core_info()
    nc, ns, lanes = info.num_cores, info.num_subcores, info.num_lanes
    nw = nc * ns
    b_per_w = batch // nw
    n_chunks = b_per_w // _CHUNK
    mesh = plsc.VectorSubcoreMesh(core_axis_name="c", subcore_axis_name="s")

    @functools.partial(
        pl.kernel,
        out_type=jax.ShapeDtypeStruct((batch,), jnp.float32),
        mesh=mesh,
        compiler_params=pltpu.CompilerParams(
            needs_layout_passes=False, use_tc_tiling_on_sc=False
        ),
        scratch_types=[
            pltpu.VMEM((n_chunks, _CHUNK), jnp.int32),
            pltpu.VMEM((n_chunks, _CHUNK), jnp.int32),
            pltpu.VMEM((n_chunks, _CHUNK, k), jnp.float32),
            pltpu.VMEM((n_chunks, _CHUNK, k), jnp.float32),
            pltpu.VMEM((n_chunks, _CHUNK), jnp.float32),
            pltpu.VMEM((n_chunks, _CHUNK), jnp.float32),
            pltpu.VMEM((b_per_w,), jnp.float32),
            pltpu.SemaphoreType.DMA,
        ],
    )
    def sc_kernel(uid_hbm, iid_hbm, bu_hbm, bv_hbm, u_hbm, v_hbm, out_hbm,
                  uid_v, iid_v, u_rows, v_rows, bu_rows, bv_rows, out_v, sem):
        wid = lax.axis_index("s") * nc + lax.axis_index("c")
        base = wid * b_per_w

        for j in range(n_chunks):
            pltpu.sync_copy(uid_hbm.at[pl.ds(base + j * _CHUNK, _CHUNK)], uid_v.at[j])
            pltpu.sync_copy(iid_hbm.at[pl.ds(base + j * _CHUNK, _CHUNK)], iid_v.at[j])

        copies = []
        for j in range(n_chunks):
            copies.append(pltpu.async_copy(u_hbm.at[uid_v.at[j]], u_rows.at[j], sem))
            copies.append(pltpu.async_copy(v_hbm.at[iid_v.at[j]], v_rows.at[j], sem))
            copies.append(pltpu.async_copy(bu_hbm.at[uid_v.at[j]], bu_rows.at[j], sem))
            copies.append(pltpu.async_copy(bv_hbm.at[iid_v.at[j]], bv_rows.at[j], sem))
        for c in copies:
            c.wait()

        iota = lax.iota(jnp.int32, lanes)
        zero_v = jnp.zeros((lanes,), jnp.int32)
        groups_per_chunk = _CHUNK // lanes

        def group_body(g, _):
            j = g // groups_per_chunk
            p0 = (g % groups_per_chunk) * lanes
            j_v = zero_v + j
            p_v = iota + p0
            acc = plsc.load_gather(bu_rows, [j_v, p_v])
            acc = acc + plsc.load_gather(bv_rows, [j_v, p_v])
            acc = acc + _MEAN
            for kk in range(k):
                k_v = zero_v + kk
                uu = plsc.load_gather(u_rows, [j_v, p_v, k_v])
                vv = plsc.load_gather(v_rows, [j_v, p_v, k_v])
                acc = acc + uu * vv
            out_v[pl.ds(g * lanes, lanes)] = acc
            return 0

        lax.fori_loop(0, b_per_w // lanes, group_body, 0)
        pltpu.sync_copy(out_v, out_hbm.at[pl.ds(base, b_per_w)])

    return sc_kernel(user_ids, item_ids, bu.reshape(-1), bv.reshape(-1), U, V)


def kernel(user_ids, item_ids, bu, bv, U, V):
    batch = user_ids.shape[0]
    k = U.shape[1]
    return _biased_mf_sc(user_ids, item_ids, bu, bv, U, V, batch=batch, k=k)


# speedup vs baseline: 1.9015x; 1.9015x over previous
import functools

import jax
import jax.numpy as jnp
from jax import lax
from jax.experimental import pallas as pl
from jax.experimental.pallas import tpu as pltpu
from jax.experimental.pallas import tpu_sc as plsc

_MEAN = 3.5
_SLABW = 512
_CHUNK = 128


@functools.partial(jax.jit, static_argnames=("batch", "k", "n"))
def _biased_mf_sc(user_ids, item_ids, bu, bv, U, V, *, batch, k, n):
    info = plsc.get_sparse_core_info()
    nc, ns, lanes = info.num_cores, info.num_subcores, info.num_lanes
    nw = nc * ns
    b_per_w = batch // nw

    blocks = (n + 127) // 128
    blocks_per_w = (blocks + nw - 1) // nw
    range_per_w = blocks_per_w * 128
    n_chunks = (range_per_w + _SLABW - 1) // _SLABW
    soff_max = ((n - _SLABW) // 128) * 128
    tail_start = (n // 128) * 128
    tail_w = n - tail_start
    n_vecs = batch // lanes
    pad_rows = batch + lanes

    mesh = plsc.VectorSubcoreMesh(core_axis_name="c", subcore_axis_name="s")

    Ut = U.T
    Vt = V.T

    row_ty = jax.ShapeDtypeStruct((pad_rows, 2 * k), jnp.float32)

    @functools.partial(
        pl.kernel,
        out_type=(row_ty, row_ty),
        mesh=mesh,
        compiler_params=pltpu.CompilerParams(needs_layout_passes=False),
        scratch_types=[
            pltpu.VMEM((batch,), jnp.int32),
            pltpu.VMEM((batch,), jnp.int32),
            pltpu.VMEM((batch,), jnp.int32),
            pltpu.VMEM((2, k, _SLABW), jnp.float32),
            pltpu.VMEM((lanes, 2 * k), jnp.float32),
            pltpu.VMEM((k, max(tail_w, 1)), jnp.float32),
            pltpu.SemaphoreType.DMA,
            pltpu.SemaphoreType.DMA,
        ],
    )
    def gather_kernel(uid_hbm, iid_hbm, ut_hbm, vt_hbm, a_hbm, b_hbm,
                      ids_all, hitlist, chunkhits, slab, rowstage, tailbuf,
                      slab_sem, sct_sem):
        wid = lax.axis_index("s") * nc + lax.axis_index("c")
        lo = wid * range_per_w
        hi = jnp.minimum(lo + range_per_w, n)
        iota = lax.iota(jnp.int32, lanes)
        zero_v = jnp.zeros((lanes,), jnp.int32)

        def run_phase(ids_hbm, t_hbm, dst_hbm):
            pltpu.sync_copy(ids_hbm, ids_all)

            def scan_body(vi, off):
                v = ids_all[pl.ds(vi * lanes, lanes)]
                m = jnp.logical_and(v >= lo, v < hi)
                plsc.store_compressed(hitlist.at[pl.ds(off, lanes)],
                                      iota + vi * lanes, mask=m)
                return off + jnp.sum(m.astype(jnp.int32))

            cnt = lax.fori_loop(0, n_vecs, scan_body, jnp.int32(0))
            n_hvecs = (cnt + lanes - 1) // lanes

            soff0 = pl.multiple_of(jnp.minimum(lo, soff_max), 128)
            pltpu.async_copy(
                t_hbm.at[pl.ds(0, k), pl.ds(soff0, _SLABW)], slab.at[0],
                slab_sem)

            def chunk_body(c, _):
                clo = lo + c * _SLABW
                soff = pl.multiple_of(jnp.minimum(clo, soff_max), 128)
                cend = jnp.minimum(clo + _SLABW, jnp.minimum(hi, tail_start))

                @pl.when(c + 1 < n_chunks)
                def _prefetch():
                    nsoff = pl.multiple_of(
                        jnp.minimum(clo + _SLABW, soff_max), 128)
                    pltpu.async_copy(
                        t_hbm.at[pl.ds(0, k), pl.ds(nsoff, _SLABW)],
                        slab.at[(c + 1) % 2], slab_sem)

                pltpu.make_async_copy(
                    t_hbm.at[pl.ds(0, k), pl.ds(0, _SLABW)], slab.at[0],
                    slab_sem).wait()
                buf = c % 2

                def hits_body(hv, coff):
                    sl = pl.ds(hv * lanes, lanes)
                    e_v = hitlist[sl]
                    valid = (iota + hv * lanes) < cnt
                    ids_v = plsc.load_gather(ids_all, [e_v], mask=valid)
                    m = jnp.logical_and(valid,
                                        jnp.logical_and(ids_v >= clo,
                                                        ids_v < cend))
                    plsc.store_compressed(chunkhits.at[pl.ds(coff, lanes)],
                                          e_v, mask=m)
                    return coff + jnp.sum(m.astype(jnp.int32))

                ccnt = lax.fori_loop(0, n_hvecs, hits_body, jnp.int32(0))
                n_gvecs = (ccnt + lanes - 1) // lanes

                def group_body(gi, _):
                    sl = pl.ds(gi * lanes, lanes)
                    e_v = chunkhits[sl]
                    valid = (iota + gi * lanes) < ccnt
                    e_safe = jnp.where(valid, e_v, batch + iota)
                    chunkhits[sl] = e_safe
                    ids_v = plsc.load_gather(ids_all, [e_v], mask=valid)
                    loc = ids_v - soff
                    for kk in range(k):
                        val = plsc.load_gather(slab, [zero_v + buf,
                                                      zero_v + kk, loc],
                                               mask=valid)
                        plsc.store_scatter(rowstage, [iota, zero_v + kk], val)
                    pltpu.async_copy(rowstage,
                                     dst_hbm.at[chunkhits.at[sl]],
                                     sct_sem).wait()
                    return 0

                lax.fori_loop(0, n_gvecs, group_body, 0)
                return 0

            lax.fori_loop(0, n_chunks, chunk_body, 0)

            if tail_w:
                @pl.when(hi == n)
                def _tail():
                    pltpu.sync_copy(
                        t_hbm.at[pl.ds(0, k), pl.ds(tail_start, tail_w)],
                        tailbuf)

                    def tail_body(hv, _):
                        sl = pl.ds(hv * lanes, lanes)
                        e_v = hitlist[sl]
                        valid = (iota + hv * lanes) < cnt
                        ids_v = plsc.load_gather(ids_all, [e_v], mask=valid)
                        m = jnp.logical_and(valid, ids_v >= tail_start)

                        @pl.when(jnp.sum(m.astype(jnp.int32)) > 0)
                        def _do():
                            loc = ids_v - tail_start
                            for kk in range(k):
                                val = plsc.load_gather(
                                    tailbuf, [zero_v + kk, loc], mask=m)
                                plsc.store_scatter(
                                    rowstage, [iota, zero_v + kk], val, mask=m)
                            chunkhits[pl.ds(0, lanes)] = jnp.where(
                                m, e_v, batch + iota)
                            pltpu.async_copy(
                                rowstage,
                                dst_hbm.at[chunkhits.at[pl.ds(0, lanes)]],
                                sct_sem).wait()

                        return 0

                    lax.fori_loop(0, n_hvecs, tail_body, 0)

        run_phase(uid_hbm, ut_hbm, a_hbm)
        run_phase(iid_hbm, vt_hbm, b_hbm)

    A, B = gather_kernel(user_ids, item_ids, Ut, Vt)

    n_bchunks = b_per_w // _CHUNK
    half = b_per_w // 2

    @functools.partial(
        pl.kernel,
        out_type=jax.ShapeDtypeStruct((batch,), jnp.float32),
        mesh=mesh,
        compiler_params=pltpu.CompilerParams(needs_layout_passes=False),
        scratch_types=[
            pltpu.VMEM((n_bchunks, _CHUNK), jnp.int32),
            pltpu.VMEM((n_bchunks, _CHUNK), jnp.int32),
            pltpu.VMEM((half, 2 * k), jnp.float32),
            pltpu.VMEM((half, 2 * k), jnp.float32),
            pltpu.VMEM((n_bchunks, _CHUNK), jnp.float32),
            pltpu.VMEM((n_bchunks, _CHUNK), jnp.float32),
            pltpu.VMEM((b_per_w,), jnp.float32),
            pltpu.SemaphoreType.DMA,
        ],
    )
    def dot_kernel(uid_hbm, iid_hbm, bu_hbm, bv_hbm, a_hbm, b_hbm, out_hbm,
                   uid_v, iid_v, a_rows, b_rows, bu_rows, bv_rows, out_v, sem):
        wid = lax.axis_index("s") * nc + lax.axis_index("c")
        base = wid * b_per_w
        iota = lax.iota(jnp.int32, lanes)
        zero_v = jnp.zeros((lanes,), jnp.int32)
        vecs_per_chunk = _CHUNK // lanes

        for j in range(n_bchunks):
            pltpu.sync_copy(uid_hbm.at[pl.ds(base + j * _CHUNK, _CHUNK)], uid_v.at[j])
            pltpu.sync_copy(iid_hbm.at[pl.ds(base + j * _CHUNK, _CHUNK)], iid_v.at[j])

        bias_copies = []
        for j in range(n_bchunks):
            bias_copies.append(pltpu.async_copy(bu_hbm.at[uid_v.at[j]], bu_rows.at[j], sem))
            bias_copies.append(pltpu.async_copy(bv_hbm.at[iid_v.at[j]], bv_rows.at[j], sem))
        for cp in bias_copies:
            cp.wait()

        for h in range(2):
            pltpu.sync_copy(a_hbm.at[pl.ds(base + h * half, half)], a_rows)
            pltpu.sync_copy(b_hbm.at[pl.ds(base + h * half, half)], b_rows)

            def group_body(g, _):
                p0 = g * lanes
                jg = (h * half + p0) // _CHUNK
                sl = pl.ds((h * half + p0) % _CHUNK, lanes)
                p_v = iota + p0
                acc = bu_rows[jg, sl] + bv_rows[jg, sl] + _MEAN
                for kk in range(k):
                    uu = plsc.load_gather(a_rows, [p_v, zero_v + kk])
                    vv = plsc.load_gather(b_rows, [p_v, zero_v + kk])
                    acc = acc + uu * vv
                out_v[pl.ds(h * half + p0, lanes)] = acc
                return 0

            lax.fori_loop(0, half // lanes, group_body, 0)

        pltpu.sync_copy(out_v, out_hbm.at[pl.ds(base, b_per_w)])

    return dot_kernel(user_ids, item_ids, bu.reshape(-1), bv.reshape(-1), A, B)


def kernel(user_ids, item_ids, bu, bv, U, V):
    batch = user_ids.shape[0]
    k = U.shape[1]
    n = U.shape[0]
    return _biased_mf_sc(user_ids, item_ids, bu, bv, U, V,
                         batch=batch, k=k, n=n)
